# baseline (speedup 1.0000x reference)
import jax
import jax.numpy as jnp
from jax.experimental import pallas as pl

_N = 10000
_E = 320000


def _mm_body(x_ref, w_ref, o_ref):
    o_ref[...] = jnp.dot(x_ref[...], w_ref[...], preferred_element_type=jnp.float32)


def _matmul(x, w):
    m, k = x.shape
    _, n = w.shape
    bm = 1000
    return pl.pallas_call(
        _mm_body,
        grid=(m // bm,),
        in_specs=[
            pl.BlockSpec((bm, k), lambda i: (i, 0)),
            pl.BlockSpec((k, n), lambda i: (0, 0)),
        ],
        out_specs=pl.BlockSpec((bm, n), lambda i: (i, 0)),
        out_shape=jax.ShapeDtypeStruct((m, n), jnp.float32),
    )(x, w)


def _conv(x, W, b, src, dst, e_w, src_norm, dst_norm):
    h = _matmul(x, W) * src_norm
    msg = jnp.take(h, src, axis=0) * e_w
    agg = jax.ops.segment_sum(msg, dst, num_segments=_N)
    return agg * dst_norm + b


def kernel(inputs, edge_index, edge_weights, W1, b1, W2, b2):
    src = edge_index[0]
    dst = edge_index[1]
    out_deg = jnp.zeros((_N,), jnp.float32).at[src].add(1.0)
    in_deg = jnp.zeros((_N,), jnp.float32).at[dst].add(1.0)
    src_norm = jax.lax.rsqrt(jnp.clip(out_deg, 1.0, None))[:, None]
    dst_norm = jax.lax.rsqrt(jnp.clip(in_deg, 1.0, None))[:, None]
    h = _conv(inputs, W1, b1, src, dst, edge_weights, src_norm, dst_norm)
    h = jax.nn.relu(h)
    reminder = jnp.concatenate([h, inputs], axis=-1)
    h = _conv(reminder, W2, b2, src, dst, edge_weights, src_norm, dst_norm)
    h = jax.nn.relu(h)
    norm = jnp.linalg.norm(h, ord=2, axis=-1, keepdims=True)
    return h / jnp.maximum(norm, 1e-12)


# speedup vs baseline: 2.3212x; 2.3212x over previous
import functools

import jax
import jax.numpy as jnp
from jax import lax
from jax.experimental import pallas as pl
from jax.experimental.pallas import tpu as pltpu
from jax.experimental.pallas import tpu_sc as plsc

_N = 10000
_E = 320000
_D = 128

_NC = 2
_NS = 16
_NW = _NC * _NS

_BATCH = 128
_EPT = 10240
_NBATCH = _EPT // _BATCH
_EPAD = _EPT * _NW
_NPAD = 10112
_RPT = _NPAD // _NS


def _mesh():
    return plsc.VectorSubcoreMesh(core_axis_name="c", subcore_axis_name="s",
                                  num_cores=_NC, num_subcores=_NS)


def _deg_body(src_hbm, dst_hbm, ones_hbm, z_hbm, out_hbm,
              idx_v, ones_v, deg_sh):
    c = lax.axis_index("c")
    s = lax.axis_index("s")
    wid = c * _NS + s
    r0 = s * _RPT

    pltpu.sync_copy(ones_hbm, ones_v)

    for hist, e_hbm in ((0, src_hbm), (1, dst_hbm)):
        pltpu.sync_copy(z_hbm.at[pl.ds(r0, _RPT)], deg_sh.at[pl.ds(r0, _RPT)])
        plsc.subcore_barrier()

        def batch(b, _):
            base = wid * _EPT + b * _BATCH
            pltpu.sync_copy(e_hbm.at[pl.ds(base, _BATCH)], idx_v)
            pltpu.sync_copy(ones_v, deg_sh.at[idx_v], add=True)
            return 0
        lax.fori_loop(0, _NBATCH, batch, 0)

        plsc.subcore_barrier()
        pltpu.sync_copy(deg_sh.at[pl.ds(r0, _RPT)],
                        out_hbm.at[c, hist, pl.ds(r0, _RPT)])
        plsc.subcore_barrier()


def _deg_kernel(src_p, dst_p, ones, z):
    f = pl.kernel(
        _deg_body,
        out_type=jax.ShapeDtypeStruct((_NC, 2, _NPAD, _D), jnp.float32),
        mesh=_mesh(),
        scratch_types=[
            pltpu.VMEM((_BATCH,), jnp.int32),
            pltpu.VMEM((_BATCH, _D), jnp.float32),
            pltpu.VMEM_SHARED((_NPAD, _D), jnp.float32),
        ],
    )
    return f(src_p, dst_p, ones, z)


def _edge_body(table_hbm, src_hbm, dst_hbm, wrep_hbm, z_hbm, out_hbm,
               idx_v, didx_v, wrep_v, rows_v, agg_sh, sem):
    c = lax.axis_index("c")
    s = lax.axis_index("s")
    wid = c * _NS + s
    r0 = s * _RPT

    pltpu.sync_copy(z_hbm.at[pl.ds(r0, _RPT)], agg_sh.at[pl.ds(r0, _RPT)])
    plsc.subcore_barrier()

    def batch(b, _):
        base = wid * _EPT + b * _BATCH
        pltpu.sync_copy(src_hbm.at[pl.ds(base, _BATCH)], idx_v)
        pltpu.sync_copy(dst_hbm.at[pl.ds(base, _BATCH)], didx_v)
        pltpu.sync_copy(wrep_hbm.at[pl.ds(base, _BATCH)], wrep_v)
        pltpu.async_copy(table_hbm.at[idx_v], rows_v, sem).wait()

        def scale(e, _):
            wv = wrep_v[e, :]
            for j in range(8):
                sl = pl.ds(j * 16, 16)
                rows_v[e, sl] = rows_v[e, sl] * wv
            return 0
        lax.fori_loop(0, _BATCH, scale, 0)

        pltpu.sync_copy(rows_v, agg_sh.at[didx_v], add=True)
        return 0
    lax.fori_loop(0, _NBATCH, batch, 0)

    plsc.subcore_barrier()
    pltpu.sync_copy(agg_sh.at[pl.ds(r0, _RPT)],
                    out_hbm.at[c, pl.ds(r0, _RPT)])


def _edge_pass(table, src_p, dst_p, wrep, z):
    f = pl.kernel(
        _edge_body,
        out_type=jax.ShapeDtypeStruct((_NC, _NPAD, _D), jnp.float32),
        mesh=_mesh(),
        scratch_types=[
            pltpu.VMEM((_BATCH,), jnp.int32),
            pltpu.VMEM((_BATCH,), jnp.int32),
            pltpu.VMEM((_BATCH, 16), jnp.float32),
            pltpu.VMEM((_BATCH, _D), jnp.float32),
            pltpu.VMEM_SHARED((_NPAD, _D), jnp.float32),
            pltpu.SemaphoreType.DMA,
        ],
    )
    return f(table, src_p, dst_p, wrep, z)


def _degnorm(dega_ref, degb_ref):
    d = jnp.sum(dega_ref[...] + degb_ref[...], axis=1, keepdims=True) * (1.0 / _D)
    return lax.rsqrt(jnp.clip(d, 1.0, None))


def _stage_a_body(x_ref, w_ref, dsa_ref, dsb_ref, o_ref):
    srcn = _degnorm(dsa_ref, dsb_ref)
    o_ref[...] = jnp.dot(x_ref[...], w_ref[...],
                         preferred_element_type=jnp.float32) * srcn


def _stage_a(x_pad, W1, deg_src_a, deg_src_b):
    return pl.pallas_call(
        _stage_a_body,
        out_shape=jax.ShapeDtypeStruct((_NPAD, _D), jnp.float32),
    )(x_pad, W1, deg_src_a, deg_src_b)


def _stage_b_body(agg_ref, dda_ref, ddb_ref, b1_ref, wh_ref, wx_ref,
                  x_ref, dsa_ref, dsb_ref, o_ref):
    dstn = _degnorm(dda_ref, ddb_ref)
    agg = agg_ref[0] + agg_ref[1]
    h1 = jnp.maximum(agg * dstn + b1_ref[...][None, :], 0.0)
    row = lax.broadcasted_iota(jnp.int32, (_NPAD, 1), 0)
    h1 = jnp.where(row < _N, h1, 0.0)
    srcn = _degnorm(dsa_ref, dsb_ref)
    o_ref[...] = (jnp.dot(h1, wh_ref[...], preferred_element_type=jnp.float32)
                  + jnp.dot(x_ref[...], wx_ref[...],
                            preferred_element_type=jnp.float32)) * srcn


def _stage_b(aggp, deg_dst_a, deg_dst_b, b1, W2h, W2x, x_pad,
             deg_src_a, deg_src_b):
    return pl.pallas_call(
        _stage_b_body,
        out_shape=jax.ShapeDtypeStruct((_NPAD, _D), jnp.float32),
    )(aggp, deg_dst_a, deg_dst_b, b1, W2h, W2x, x_pad, deg_src_a, deg_src_b)


def _stage_c_body(agg_ref, dda_ref, ddb_ref, b2_ref, o_ref):
    dstn = _degnorm(dda_ref, ddb_ref)
    agg = agg_ref[0] + agg_ref[1]
    h2 = jnp.maximum(agg * dstn + b2_ref[...][None, :], 0.0)
    nrm = jnp.sqrt(jnp.sum(h2 * h2, axis=-1, keepdims=True))
    o_ref[...] = h2 / jnp.maximum(nrm, 1e-12)


def _stage_c(aggp, deg_dst_a, deg_dst_b, b2):
    return pl.pallas_call(
        _stage_c_body,
        out_shape=jax.ShapeDtypeStruct((_NPAD, _D), jnp.float32),
    )(aggp, deg_dst_a, deg_dst_b, b2)


def kernel(inputs, edge_index, edge_weights, W1, b1, W2, b2):
    src = edge_index[0]
    dst = edge_index[1]
    w = edge_weights[:, 0]

    npad_fill = jnp.full((_EPAD - _E,), _N, jnp.int32)
    src_p = jnp.concatenate([src, npad_fill])
    dst_p = jnp.concatenate([dst, npad_fill])
    w_p = jnp.concatenate([w, jnp.zeros((_EPAD - _E,), jnp.float32)])
    wrep = jnp.broadcast_to(w_p[:, None], (_EPAD, 16))

    x_pad = jnp.pad(inputs, ((0, _NPAD - _N), (0, 0)))
    z = jnp.zeros((_NPAD, _D), jnp.float32)
    ones = jnp.ones((_BATCH, _D), jnp.float32)

    deg = _deg_kernel(src_p, dst_p, ones, z)
    deg_src_a, deg_dst_a = deg[0, 0], deg[0, 1]
    deg_src_b, deg_dst_b = deg[1, 0], deg[1, 1]

    table1 = _stage_a(x_pad, W1, deg_src_a, deg_src_b)
    agg1 = _edge_pass(table1, src_p, dst_p, wrep, z)
    table2 = _stage_b(agg1, deg_dst_a, deg_dst_b, b1, W2[:_D], W2[_D:],
                      x_pad, deg_src_a, deg_src_b)
    agg2 = _edge_pass(table2, src_p, dst_p, wrep, z)
    out = _stage_c(agg2, deg_dst_a, deg_dst_b, b2)
    return out[:_N]


# speedup vs baseline: 8.1568x; 3.5141x over previous
import jax
import jax.numpy as jnp
import numpy as np
from jax import lax
from jax.experimental import pallas as pl
from jax.experimental.pallas import tpu as pltpu
from jax.experimental.pallas import tpu_sc as plsc

_N = 10000
_E = 320000
_D = 128

_NC = 2
_NS = 16
_NW = _NC * _NS

_BATCH = 128
_EPT = 10240
_EREAL = _E // _NW
_NBATCH = _EPT // _BATCH
_EB = 64
_ENB = _EPT // _EB
_EPAD = _EPT * _NW
_NPAD = 10112
_RPT = _NPAD // _NS
_NSLOT = 4


def _mesh():
    return plsc.VectorSubcoreMesh(core_axis_name="c", subcore_axis_name="s",
                                  num_cores=_NC, num_subcores=_NS)


def _deg_body(src_hbm, dst_hbm, ones_hbm, z_hbm, out_hbm,
              idx_all, ones_v, deg_sh, sem):
    c = lax.axis_index("c")
    s = lax.axis_index("s")
    wid = c * _NS + s
    r0 = s * _RPT

    pltpu.sync_copy(ones_hbm, ones_v)

    for hist, e_hbm in ((0, src_hbm), (1, dst_hbm)):
        pltpu.sync_copy(e_hbm.at[wid], idx_all)
        pltpu.sync_copy(z_hbm.at[pl.ds(r0, _RPT)], deg_sh.at[pl.ds(r0, _RPT)])
        plsc.subcore_barrier()

        def issue(b, _):
            pltpu.async_copy(ones_v, deg_sh.at[idx_all.at[b]], sem, add=True)

            @pl.when(b >= 16)
            def _():
                pltpu.make_async_copy(ones_v, deg_sh.at[idx_all.at[b]],
                                      sem).wait()
            return 0
        lax.fori_loop(0, _NBATCH, issue, 0)

        def drain(b, _):
            pltpu.make_async_copy(ones_v, deg_sh.at[idx_all.at[b]], sem).wait()
            return 0
        lax.fori_loop(0, 16, drain, 0)

        plsc.subcore_barrier()
        pltpu.sync_copy(deg_sh.at[pl.ds(r0, _RPT)],
                        out_hbm.at[c, hist, pl.ds(r0, _RPT)])
        plsc.subcore_barrier()


def _deg_kernel(src3, dst3, ones, z):
    f = pl.kernel(
        _deg_body,
        out_type=jax.ShapeDtypeStruct((_NC, 2, _NPAD, _D), jnp.float32),
        mesh=_mesh(),
        scratch_types=[
            pltpu.VMEM((_NBATCH, _BATCH), jnp.int32),
            pltpu.VMEM((_BATCH, _D), jnp.float32),
            pltpu.VMEM_SHARED((_NPAD, _D), jnp.float32),
            pltpu.SemaphoreType.DMA,
        ],
    )
    return f(src3, dst3, ones, z)


def _edge_body(table_hbm, src_hbm, dst_hbm, wrep_hbm, z_hbm, out_hbm,
               sidx, didx, wrep_v, rows_v, agg_sh, isem, gsem, ssem):
    c = lax.axis_index("c")
    s = lax.axis_index("s")
    wid = c * _NS + s
    r0 = s * _RPT

    pltpu.sync_copy(z_hbm.at[pl.ds(r0, _RPT)], agg_sh.at[pl.ds(r0, _RPT)])
    plsc.subcore_barrier()

    def idx_start(q, b):
        base = wid * _EPT + b * _EB
        wrow = pl.multiple_of(base // 8, 8)
        pltpu.async_copy(src_hbm.at[pl.ds(base, _EB)], sidx[q], isem[q])
        pltpu.async_copy(dst_hbm.at[pl.ds(base, _EB)], didx[q], isem[q])
        pltpu.async_copy(wrep_hbm.at[pl.ds(wrow, _EB * 16 // 128)],
                         wrep_v[q], isem[q])

    def idx_wait(q, b):
        base = wid * _EPT + b * _EB
        wrow = pl.multiple_of(base // 8, 8)
        pltpu.make_async_copy(src_hbm.at[pl.ds(base, _EB)], sidx[q],
                              isem[q]).wait()
        pltpu.make_async_copy(dst_hbm.at[pl.ds(base, _EB)], didx[q],
                              isem[q]).wait()
        pltpu.make_async_copy(wrep_hbm.at[pl.ds(wrow, _EB * 16 // 128)],
                              wrep_v[q], isem[q]).wait()

    def gather_start(q):
        pltpu.async_copy(table_hbm.at[sidx[q]], rows_v[q], gsem[q])

    def gather_wait(q):
        pltpu.make_async_copy(table_hbm.at[sidx[q]], rows_v[q],
                              gsem[q]).wait()

    def scatter_start(q):
        pltpu.async_copy(rows_v[q], agg_sh.at[didx[q]], ssem[q], add=True)

    def scatter_wait(q):
        pltpu.make_async_copy(rows_v[q], agg_sh.at[didx[q]], ssem[q]).wait()

    for i in range(3):
        idx_start(i, i)
    for i in range(2):
        idx_wait(i, i)
        gather_start(i)

    def jbody(j, _):
        for i in range(_NSLOT):
            b = _NSLOT * j + i
            slot = i

            gather_wait(slot)

            @plsc.parallel_loop(0, _EB, unroll=2)
            def _(e):
                wv = wrep_v[slot][e // 8, pl.ds((e % 8) * 16, 16)]
                for jj in range(8):
                    sl = pl.ds(jj * 16, 16)
                    rows_v[slot][e, sl] = rows_v[slot][e, sl] * wv

            scatter_start(slot)

            q3 = (i + 3) % _NSLOT

            @pl.when(b + 3 < _ENB)
            def _():
                @pl.when(b >= 1)
                def _():
                    scatter_wait(q3)
                idx_start(q3, b + 3)

            q2 = (i + 2) % _NSLOT

            @pl.when(b + 2 < _ENB)
            def _():
                idx_wait(q2, b + 2)
                gather_start(q2)
        return 0
    lax.fori_loop(0, _ENB // _NSLOT, jbody, 0)

    for i in range(_NSLOT):
        scatter_wait(i)

    plsc.subcore_barrier()
    pltpu.sync_copy(agg_sh.at[pl.ds(r0, _RPT)],
                    out_hbm.at[c, pl.ds(r0, _RPT)])


def _edge_pass(table, src_f, dst_f, wrep, z):
    f = pl.kernel(
        _edge_body,
        out_type=jax.ShapeDtypeStruct((_NC, _NPAD, _D), jnp.float32),
        mesh=_mesh(),
        scratch_types=[
            [pltpu.VMEM((_EB,), jnp.int32) for _ in range(_NSLOT)],
            [pltpu.VMEM((_EB,), jnp.int32) for _ in range(_NSLOT)],
            [pltpu.VMEM((_EB * 16 // 128, _D), jnp.float32) for _ in range(_NSLOT)],
            [pltpu.VMEM((_EB, _D), jnp.float32) for _ in range(_NSLOT)],
            pltpu.VMEM_SHARED((_NPAD, _D), jnp.float32),
            [pltpu.SemaphoreType.DMA for _ in range(_NSLOT)],
            [pltpu.SemaphoreType.DMA for _ in range(_NSLOT)],
            [pltpu.SemaphoreType.DMA for _ in range(_NSLOT)],
        ],
    )
    return f(table, src_f, dst_f, wrep, z)


def _degnorm(dega_ref, degb_ref):
    d = jnp.sum(dega_ref[...] + degb_ref[...], axis=1, keepdims=True) * (1.0 / _D)
    return lax.rsqrt(jnp.clip(d, 1.0, None))


def _stage_a_body(x_ref, w_ref, dsa_ref, dsb_ref, o_ref):
    srcn = _degnorm(dsa_ref, dsb_ref)
    o_ref[...] = jnp.dot(x_ref[...], w_ref[...],
                         preferred_element_type=jnp.float32) * srcn


def _stage_a(x_pad, W1, deg_src_a, deg_src_b):
    return pl.pallas_call(
        _stage_a_body,
        out_shape=jax.ShapeDtypeStruct((_NPAD, _D), jnp.float32),
    )(x_pad, W1, deg_src_a, deg_src_b)


def _stage_b_body(agg_ref, dda_ref, ddb_ref, b1_ref, wh_ref, wx_ref,
                  x_ref, dsa_ref, dsb_ref, o_ref):
    dstn = _degnorm(dda_ref, ddb_ref)
    agg = agg_ref[0] + agg_ref[1]
    h1 = jnp.maximum(agg * dstn + b1_ref[...][None, :], 0.0)
    row = lax.broadcasted_iota(jnp.int32, (_NPAD, 1), 0)
    h1 = jnp.where(row < _N, h1, 0.0)
    srcn = _degnorm(dsa_ref, dsb_ref)
    o_ref[...] = (jnp.dot(h1, wh_ref[...], preferred_element_type=jnp.float32)
                  + jnp.dot(x_ref[...], wx_ref[...],
                            preferred_element_type=jnp.float32)) * srcn


def _stage_b(aggp, deg_dst_a, deg_dst_b, b1, W2h, W2x, x_pad,
             deg_src_a, deg_src_b):
    return pl.pallas_call(
        _stage_b_body,
        out_shape=jax.ShapeDtypeStruct((_NPAD, _D), jnp.float32),
    )(aggp, deg_dst_a, deg_dst_b, b1, W2h, W2x, x_pad, deg_src_a, deg_src_b)


def _stage_c_body(agg_ref, dda_ref, ddb_ref, b2_ref, o_ref):
    dstn = _degnorm(dda_ref, ddb_ref)
    agg = agg_ref[0] + agg_ref[1]
    h2 = jnp.maximum(agg * dstn + b2_ref[...][None, :], 0.0)
    nrm = jnp.sqrt(jnp.sum(h2 * h2, axis=-1, keepdims=True))
    o_ref[...] = h2 / jnp.maximum(nrm, 1e-12)


def _stage_c(aggp, deg_dst_a, deg_dst_b, b2):
    return pl.pallas_call(
        _stage_c_body,
        out_shape=jax.ShapeDtypeStruct((_NPAD, _D), jnp.float32),
    )(aggp, deg_dst_a, deg_dst_b, b2)


_PAD_IDX = np.broadcast_to(10000 + (np.arange(_EPT - _EREAL) % (_NPAD - _N)),
                           (_NW, _EPT - _EREAL)).astype(np.int32)


def kernel(inputs, edge_index, edge_weights, W1, b1, W2, b2):
    src = edge_index[0]
    dst = edge_index[1]
    w = edge_weights[:, 0]

    src_t = jnp.concatenate([src.reshape(_NW, _EREAL), _PAD_IDX], axis=1)
    dst_t = jnp.concatenate([dst.reshape(_NW, _EREAL), _PAD_IDX], axis=1)
    src3 = src_t.reshape(_NW, _NBATCH, _BATCH)
    dst3 = dst_t.reshape(_NW, _NBATCH, _BATCH)
    src_f = src_t.reshape(-1)
    dst_f = dst_t.reshape(-1)
    w_p = jnp.concatenate(
        [w.reshape(_NW, _EREAL),
         jnp.zeros((_NW, _EPT - _EREAL), jnp.float32)], axis=1).reshape(-1)
    wrep = jnp.broadcast_to(w_p[:, None], (_EPAD, 16)).reshape(_EPAD // 8, _D)

    x_pad = jnp.pad(inputs, ((0, _NPAD - _N), (0, 0)))
    z = jnp.zeros((_NPAD, _D), jnp.float32)
    ones = jnp.ones((_BATCH, _D), jnp.float32)

    deg = _deg_kernel(src3, dst3, ones, z)
    deg_src_a, deg_dst_a = deg[0, 0], deg[0, 1]
    deg_src_b, deg_dst_b = deg[1, 0], deg[1, 1]

    table1 = _stage_a(x_pad, W1, deg_src_a, deg_src_b)
    agg1 = _edge_pass(table1, src_f, dst_f, wrep, z)
    table2 = _stage_b(agg1, deg_dst_a, deg_dst_b, b1, W2[:_D], W2[_D:],
                      x_pad, deg_src_a, deg_src_b)
    agg2 = _edge_pass(table2, src_f, dst_f, wrep, z)
    out = _stage_c(agg2, deg_dst_a, deg_dst_b, b2)
    return out[:_N]
